# baseline (speedup 1.0000x reference)
import jax
import jax.numpy as jnp
from jax.experimental import pallas as pl
from jax.experimental.pallas import tpu as pltpu

_B, _N, _D = 32, 100, 128
_NC = 4
_G = _B // _NC
_ROWS = _G * _N
_INV_DEG = 1.0 / (_N + 1)


def _encoder_kernel(x_ref, wi_ref, bi_ref, w0_ref, b0_ref, w1_ref, b1_ref,
                    w2_ref, b2_ref, upd_ref, nf_ref):
    nf = jnp.dot(x_ref[0], wi_ref[...], preferred_element_type=jnp.float32)
    nf = nf + bi_ref[...]

    row_graph = jax.lax.broadcasted_iota(jnp.int32, (_G, _ROWS), 1) // _N
    graph_id = jax.lax.broadcasted_iota(jnp.int32, (_G, _ROWS), 0)
    sel = jnp.where(row_graph == graph_id, _INV_DEG, 0.0)
    row_graph_t = jax.lax.broadcasted_iota(jnp.int32, (_ROWS, _G), 0) // _N
    graph_id_t = jax.lax.broadcasted_iota(jnp.int32, (_ROWS, _G), 1)
    sel_t = jnp.where(row_graph_t == graph_id_t, 1.0, 0.0)

    h = nf
    for w_ref, b_ref, relu in ((w0_ref, b0_ref, True),
                               (w1_ref, b1_ref, True),
                               (w2_ref, b2_ref, False)):
        xw = jnp.dot(h, w_ref[...], preferred_element_type=jnp.float32)
        sg = jnp.dot(sel, xw, preferred_element_type=jnp.float32)
        bsum = jnp.dot(sel_t, sg, preferred_element_type=jnp.float32)
        h = xw * _INV_DEG + bsum + b_ref[...]
        if relu:
            h = jnp.maximum(h, 0.0)
    m = jnp.max(h, axis=1, keepdims=True)
    e = h - m
    lse = jnp.log(jnp.sum(jnp.exp(e), axis=1, keepdims=True))
    h = e - lse
    upd_ref[0] = h + nf
    nf_ref[0] = nf


def kernel(x, W_init, b_init, W0, b0, W1, b1, W2, b2):
    x2 = x.reshape(_NC, _ROWS, 2)
    b_init = b_init.reshape(1, _D)
    b0 = b0.reshape(1, _D)
    b1 = b1.reshape(1, _D)
    b2 = b2.reshape(1, _D)

    full = lambda shape: pl.BlockSpec(shape, lambda i: (0,) * len(shape))
    out_shape = jax.ShapeDtypeStruct((_NC, _ROWS, _D), jnp.float32)
    update, node_feature = pl.pallas_call(
        _encoder_kernel,
        grid=(_NC,),
        in_specs=[
            pl.BlockSpec((1, _ROWS, 2), lambda i: (i, 0, 0)),
            full((2, _D)),
            full((1, _D)),
            full((_D, _D)),
            full((1, _D)),
            full((_D, _D)),
            full((1, _D)),
            full((_D, _D)),
            full((1, _D)),
        ],
        out_specs=[
            pl.BlockSpec((1, _ROWS, _D), lambda i: (i, 0, 0)),
            pl.BlockSpec((1, _ROWS, _D), lambda i: (i, 0, 0)),
        ],
        out_shape=[out_shape, out_shape],
        compiler_params=pltpu.CompilerParams(
            dimension_semantics=("parallel",)),
    )(x2, W_init, b_init, W0, b0, W1, b1, W2, b2)
    return (update.reshape(_B, _N, _D), node_feature.reshape(_B, _N, _D))


# speedup vs baseline: 1.0010x; 1.0010x over previous
import jax
import jax.numpy as jnp
from jax.experimental import pallas as pl

_B, _N, _D = 32, 100, 128
_BN = _B * _N
_INV_DEG = 1.0 / (_N + 1)


def _encoder_kernel(x_ref, wi_ref, bi_ref, w0_ref, b0_ref, w1_ref, b1_ref,
                    w2_ref, b2_ref, upd_ref, nf_ref):
    nf = jnp.dot(x_ref[...], wi_ref[...], preferred_element_type=jnp.float32)
    nf = nf + bi_ref[...]

    row_graph = jax.lax.broadcasted_iota(jnp.int32, (_B, _BN), 1) // _N
    graph_id = jax.lax.broadcasted_iota(jnp.int32, (_B, _BN), 0)
    sel = jnp.where(row_graph == graph_id, 1.0, 0.0)
    row_graph_t = jax.lax.broadcasted_iota(jnp.int32, (_BN, _B), 0) // _N
    graph_id_t = jax.lax.broadcasted_iota(jnp.int32, (_BN, _B), 1)
    sel_t = jnp.where(row_graph_t == graph_id_t, 1.0, 0.0)

    h = nf
    for w_ref, b_ref, relu in ((w0_ref, b0_ref, True),
                               (w1_ref, b1_ref, True),
                               (w2_ref, b2_ref, False)):
        xw = jnp.dot(h.astype(jnp.bfloat16), w_ref[...],
                     preferred_element_type=jnp.float32)
        sg = jnp.dot(sel, xw, preferred_element_type=jnp.float32)
        bsum = jnp.dot(sel_t, sg, preferred_element_type=jnp.float32)
        h = xw + bsum + b_ref[...]
        if relu:
            h = jnp.maximum(h, 0.0)
    m = jnp.max(h, axis=1, keepdims=True)
    e = h - m
    lse = jnp.log(jnp.sum(jnp.exp(e), axis=1, keepdims=True))
    h = e - lse
    upd_ref[...] = h + nf
    nf_ref[...] = nf


def kernel(x, W_init, b_init, W0, b0, W1, b1, W2, b2):
    x2 = x.reshape(_BN, 2)
    b_init = b_init.reshape(1, _D)
    b0 = b0.reshape(1, _D)
    b1 = b1.reshape(1, _D)
    b2 = b2.reshape(1, _D)
    W0 = (W0 * _INV_DEG).astype(jnp.bfloat16)
    W1 = (W1 * _INV_DEG).astype(jnp.bfloat16)
    W2 = (W2 * _INV_DEG).astype(jnp.bfloat16)

    out_shape = jax.ShapeDtypeStruct((_BN, _D), jnp.float32)
    update, node_feature = pl.pallas_call(
        _encoder_kernel,
        out_shape=[out_shape, out_shape],
    )(x2, W_init, b_init, W0, b0, W1, b1, W2, b2)
    return (update.reshape(_B, _N, _D), node_feature.reshape(_B, _N, _D))


# speedup vs baseline: 1.2019x; 1.2007x over previous
import jax
import jax.numpy as jnp
from jax.experimental import pallas as pl

_B, _N, _D = 32, 100, 128
_BN = _B * _N
_INV_DEG = 1.0 / (_N + 1)


def _encoder_kernel(x_ref, wi_ref, bi_ref, w0_ref, b0_ref, w1_ref, b1_ref,
                    w2_ref, b2_ref, upd_ref, nf_ref):
    nf = jnp.dot(x_ref[...], wi_ref[...], preferred_element_type=jnp.float32)
    nf = nf + bi_ref[...]

    row_graph = jax.lax.broadcasted_iota(jnp.int32, (_B, _BN), 1) // _N
    graph_id = jax.lax.broadcasted_iota(jnp.int32, (_B, _BN), 0)
    sel = jnp.where(row_graph == graph_id, 1.0, 0.0)
    row_graph_t = jax.lax.broadcasted_iota(jnp.int32, (_BN, _B), 0) // _N
    graph_id_t = jax.lax.broadcasted_iota(jnp.int32, (_BN, _B), 1)
    sel_t = jnp.where(row_graph_t == graph_id_t, 1.0, 0.0)

    h = nf
    for w_ref, b_ref, relu in ((w0_ref, b0_ref, True),
                               (w1_ref, b1_ref, True),
                               (w2_ref, b2_ref, False)):
        xw = jnp.dot(h, w_ref[...] * _INV_DEG,
                     preferred_element_type=jnp.float32)
        sg = jnp.dot(sel, xw, preferred_element_type=jnp.float32) + b_ref[...]
        h = xw + jnp.dot(sel_t, sg, preferred_element_type=jnp.float32)
        if relu:
            h = jnp.maximum(h, 0.0)
    m = jnp.max(h, axis=1, keepdims=True)
    e = h - m
    lse = jnp.log(jnp.sum(jnp.exp(e), axis=1, keepdims=True))
    upd_ref[...] = e + (nf - lse)
    nf_ref[...] = nf


def kernel(x, W_init, b_init, W0, b0, W1, b1, W2, b2):
    x2 = x.reshape(_BN, 2)
    b_init = b_init.reshape(1, _D)
    b0 = b0.reshape(1, _D)
    b1 = b1.reshape(1, _D)
    b2 = b2.reshape(1, _D)

    out_shape = jax.ShapeDtypeStruct((_BN, _D), jnp.float32)
    update, node_feature = pl.pallas_call(
        _encoder_kernel,
        out_shape=[out_shape, out_shape],
    )(x2, W_init, b_init, W0, b0, W1, b1, W2, b2)
    return (update.reshape(_B, _N, _D), node_feature.reshape(_B, _N, _D))
